# baseline (speedup 1.0000x reference)
import functools

import jax
import jax.numpy as jnp
from jax import lax
from jax.experimental import pallas as pl
from jax.experimental.pallas import tpu as pltpu
from jax.experimental.pallas import tpu_sc as plsc

N = 10000
E = 320000
D = 128
G = 64
C = 16

NC = 2
NS = 16
NW = NC * NS
EP = E // NW
K = 40
NCHUNK = EP // K
NP = 10240
RP = NP // NS

def _make_mesh():
    return plsc.VectorSubcoreMesh(core_axis_name="c", subcore_axis_name="s",
                                  num_cores=NC, num_subcores=NS)


@functools.cache
def _get_sc_degree():
  return pl.kernel(
    _sc_degree_body,
    out_type=jax.ShapeDtypeStruct((NC * NP, D), jnp.float32),
    mesh=_make_mesh(),
    scratch_types=[
        pltpu.VMEM((K,), jnp.int32),
        pltpu.VMEM((K, D), jnp.float32),
        pltpu.VMEM_SHARED((NP, D), jnp.float32),
        pltpu.SemaphoreType.DMA,
    ],
  )


def _sc_degree_body(col_hbm, ones_hbm, zeros_hbm, degp_hbm, colv, onesv, dega, sem):
    c = lax.axis_index("c")
    s = lax.axis_index("s")
    pltpu.sync_copy(zeros_hbm.at[pl.ds(s * RP, RP)], dega.at[pl.ds(s * RP, RP)])
    pltpu.sync_copy(ones_hbm, onesv)
    plsc.subcore_barrier()
    base = (c * NS + s) * EP

    def body(i, carry):
        off = base + i * K
        pltpu.sync_copy(col_hbm.at[pl.ds(off, K)], colv)
        pltpu.sync_copy(onesv, dega.at[colv], add=True)
        return carry

    lax.fori_loop(0, NCHUNK, body, 0)
    plsc.subcore_barrier()
    pltpu.sync_copy(dega.at[pl.ds(s * RP, RP)],
                    degp_hbm.at[pl.ds(c * NP + s * RP, RP)])


NB = 5
NG = NCHUNK // NB


@functools.cache
def _get_sc_gather_scatter():
  return pl.kernel(
    _sc_gather_scatter_body,
    out_type=jax.ShapeDtypeStruct((NC * NP, D), jnp.float32),
    mesh=_make_mesh(),
    scratch_types=[
        pltpu.VMEM((EP,), jnp.int32),
        pltpu.VMEM((EP,), jnp.int32),
        [pltpu.VMEM((K, D), jnp.float32)] * NB,
        [pltpu.SemaphoreType.DMA] * NB,
        [pltpu.SemaphoreType.DMA] * NB,
        pltpu.VMEM_SHARED((NP, D), jnp.float32),
    ],
  )


def _sc_gather_scatter_body(y_hbm, row_hbm, col_hbm, zeros_hbm, accp_hbm,
                            idxr, idxc, bufs, sem_g, sem_s, acc):
    c = lax.axis_index("c")
    s = lax.axis_index("s")
    base = (c * NS + s) * EP
    pltpu.sync_copy(row_hbm.at[pl.ds(base, EP)], idxr)
    pltpu.sync_copy(col_hbm.at[pl.ds(base, EP)], idxc)
    pltpu.sync_copy(zeros_hbm.at[pl.ds(s * RP, RP)], acc.at[pl.ds(s * RP, RP)])
    plsc.subcore_barrier()

    def rnd(j, carry):
        gds = [pltpu.async_copy(
            y_hbm.at[idxr.at[pl.ds((j * NB + b) * K, K)]],
            bufs[b], sem_g[b]) for b in range(NB)]
        for b in range(NB):
            gds[b].wait()
            pltpu.sync_copy(bufs[b],
                            acc.at[idxc.at[pl.ds((j * NB + b) * K, K)]],
                            add=True)
        return carry

    lax.fori_loop(0, NG, rnd, 0)
    plsc.subcore_barrier()
    pltpu.sync_copy(acc.at[pl.ds(s * RP, RP)],
                    accp_hbm.at[pl.ds(c * NP + s * RP, RP)])


def _dinv_from(degp_ref):
    deg = degp_ref[0:N, 0:1] + degp_ref[NP:NP + N, 0:1] + 1.0
    return lax.rsqrt(deg)


def _tc1_body(x_ref, w1_ref, degp_ref, y_ref):
    dinv = _dinv_from(degp_ref)
    xw = jnp.dot(x_ref[...], w1_ref[...], preferred_element_type=jnp.float32)
    y_ref[...] = xw * dinv


def _tc2_body(accp_ref, y1_ref, degp_ref, b1_ref, w2_ref, y2_ref):
    dinv = _dinv_from(degp_ref)
    agg = accp_ref[0:N, :] + accp_ref[NP:NP + N, :] + y1_ref[...]
    h = jnp.maximum(agg * dinv + b1_ref[...], 0.0)
    y2_ref[...] = jnp.dot(h, w2_ref[...], preferred_element_type=jnp.float32) * dinv


def _tc3_body(accp_ref, y2_ref, degp_ref, b2_ref, batch_ref, wc_ref, bc_ref,
              out_ref):
    dinv = _dinv_from(degp_ref)
    agg = accp_ref[0:N, :] + accp_ref[NP:NP + N, :] + y2_ref[...]
    h = jnp.maximum(agg * dinv + b2_ref[...], 0.0)
    gids = lax.broadcasted_iota(jnp.int32, (N, G), 1)
    onehot = (batch_ref[...] == gids).astype(jnp.float32)
    sums = lax.dot_general(onehot, h, (((0,), (0,)), ((), ())),
                           preferred_element_type=jnp.float32)
    counts = lax.dot_general(onehot, jnp.ones((N, 1), jnp.float32),
                             (((0,), (0,)), ((), ())),
                             preferred_element_type=jnp.float32)
    pooled = sums / jnp.maximum(counts, 1.0)
    logits = jnp.dot(pooled, wc_ref[...],
                     preferred_element_type=jnp.float32) + bc_ref[...]
    m = jnp.max(logits, axis=1, keepdims=True)
    ex = jnp.exp(logits - m)
    out_ref[...] = ex / jnp.sum(ex, axis=1, keepdims=True)


_tc1 = pl.pallas_call(
    _tc1_body, out_shape=jax.ShapeDtypeStruct((N, D), jnp.float32))
_tc2 = pl.pallas_call(
    _tc2_body, out_shape=jax.ShapeDtypeStruct((N, D), jnp.float32))
_tc3 = pl.pallas_call(
    _tc3_body, out_shape=jax.ShapeDtypeStruct((G, C), jnp.float32))


def kernel(x, edge_index, batch, W1, b1, W2, b2, Wc, bc):
    row = edge_index[0]
    col = edge_index[1]
    col_flat = col
    zeros128 = jnp.zeros((NP, D), jnp.float32)
    ones128 = jnp.ones((K, D), jnp.float32)

    degp = _get_sc_degree()(col_flat, ones128, zeros128)
    y1 = _tc1(x, W1, degp)
    accp1 = _get_sc_gather_scatter()(y1, row, col, zeros128)
    y2 = _tc2(accp1, y1, degp, b1.reshape(1, D), W2)
    accp2 = _get_sc_gather_scatter()(y2, row, col, zeros128)
    return _tc3(accp2, y2, degp, b2.reshape(1, D), batch.reshape(N, 1),
                Wc, bc.reshape(1, C))


# speedup vs baseline: 1.1824x; 1.1824x over previous
import functools

import jax
import jax.numpy as jnp
from jax import lax
from jax.experimental import pallas as pl
from jax.experimental.pallas import tpu as pltpu
from jax.experimental.pallas import tpu_sc as plsc

N = 10000
E = 320000
D = 128
G = 64
C = 16

NC = 2
NS = 16
NW = NC * NS
EP = E // NW
K = 40
NCHUNK = EP // K
NP = 10240
RP = NP // NS

def _make_mesh():
    return plsc.VectorSubcoreMesh(core_axis_name="c", subcore_axis_name="s",
                                  num_cores=NC, num_subcores=NS)


@functools.cache
def _get_sc_degree():
  return pl.kernel(
    _sc_degree_body,
    out_type=jax.ShapeDtypeStruct((NC * NP, D), jnp.float32),
    mesh=_make_mesh(),
    scratch_types=[
        pltpu.VMEM((K,), jnp.int32),
        pltpu.VMEM((K, D), jnp.float32),
        pltpu.VMEM_SHARED((NP, D), jnp.float32),
        pltpu.SemaphoreType.DMA,
    ],
  )


def _sc_degree_body(col_hbm, ones_hbm, zeros_hbm, degp_hbm, colv, onesv, dega, sem):
    c = lax.axis_index("c")
    s = lax.axis_index("s")
    pltpu.sync_copy(zeros_hbm.at[pl.ds(s * RP, RP)], dega.at[pl.ds(s * RP, RP)])
    pltpu.sync_copy(ones_hbm, onesv)
    plsc.subcore_barrier()
    base = (c * NS + s) * EP

    def body(i, carry):
        off = base + i * K
        pltpu.sync_copy(col_hbm.at[pl.ds(off, K)], colv)
        pltpu.sync_copy(onesv, dega.at[colv], add=True)
        return carry

    lax.fori_loop(0, NCHUNK, body, 0)
    plsc.subcore_barrier()
    pltpu.sync_copy(dega.at[pl.ds(s * RP, RP)],
                    degp_hbm.at[pl.ds(c * NP + s * RP, RP)])


NB = 5
NG = NCHUNK // NB


@functools.cache
def _get_sc_gather_scatter():
  return pl.kernel(
    _sc_gather_scatter_body,
    out_type=jax.ShapeDtypeStruct((NC * NP, D), jnp.float32),
    mesh=_make_mesh(),
    scratch_types=[
        pltpu.VMEM((EP,), jnp.int32),
        pltpu.VMEM((EP,), jnp.int32),
        [pltpu.VMEM((K, D), jnp.float32)] * NB,
        [pltpu.SemaphoreType.DMA] * NB,
        [pltpu.SemaphoreType.DMA] * NB,
        pltpu.VMEM_SHARED((NP, D), jnp.float32),
    ],
  )


def _sc_gather_scatter_body(y_hbm, row_hbm, col_hbm, zeros_hbm, accp_hbm,
                            idxr, idxc, bufs, sem_g, sem_s, acc):
    c = lax.axis_index("c")
    s = lax.axis_index("s")
    base = (c * NS + s) * EP
    pltpu.sync_copy(row_hbm.at[pl.ds(base, EP)], idxr)
    pltpu.sync_copy(col_hbm.at[pl.ds(base, EP)], idxc)
    pltpu.sync_copy(zeros_hbm.at[pl.ds(s * RP, RP)], acc.at[pl.ds(s * RP, RP)])
    plsc.subcore_barrier()

    def rnd(j, carry):
        gds = []
        for b in range(NB):
            @pl.when(j > 0)
            def _():
                pltpu.make_async_copy(bufs[b], acc.at[pl.ds(0, K)],
                                      sem_s[b]).wait()
            gds.append(pltpu.async_copy(
                y_hbm.at[idxr.at[pl.ds((j * NB + b) * K, K)]],
                bufs[b], sem_g[b]))
        for b in range(NB):
            gds[b].wait()
            pltpu.async_copy(bufs[b],
                             acc.at[idxc.at[pl.ds((j * NB + b) * K, K)]],
                             sem_s[b], add=True)
        return carry

    lax.fori_loop(0, NG, rnd, 0)
    for b in range(NB):
        pltpu.make_async_copy(bufs[b], acc.at[pl.ds(0, K)], sem_s[b]).wait()
    plsc.subcore_barrier()
    pltpu.sync_copy(acc.at[pl.ds(s * RP, RP)],
                    accp_hbm.at[pl.ds(c * NP + s * RP, RP)])


def _dinv_from(degp_ref):
    deg = degp_ref[0:N, 0:1] + degp_ref[NP:NP + N, 0:1] + 1.0
    return lax.rsqrt(deg)


def _tc1_body(x_ref, w1_ref, degp_ref, y_ref):
    dinv = _dinv_from(degp_ref)
    xw = jnp.dot(x_ref[...], w1_ref[...], preferred_element_type=jnp.float32)
    y_ref[...] = xw * dinv


def _tc2_body(accp_ref, y1_ref, degp_ref, b1_ref, w2_ref, y2_ref):
    dinv = _dinv_from(degp_ref)
    agg = accp_ref[0:N, :] + accp_ref[NP:NP + N, :] + y1_ref[...]
    h = jnp.maximum(agg * dinv + b1_ref[...], 0.0)
    y2_ref[...] = jnp.dot(h, w2_ref[...], preferred_element_type=jnp.float32) * dinv


def _tc3_body(accp_ref, y2_ref, degp_ref, b2_ref, batch_ref, wc_ref, bc_ref,
              out_ref):
    dinv = _dinv_from(degp_ref)
    agg = accp_ref[0:N, :] + accp_ref[NP:NP + N, :] + y2_ref[...]
    h = jnp.maximum(agg * dinv + b2_ref[...], 0.0)
    gids = lax.broadcasted_iota(jnp.int32, (N, G), 1)
    onehot = (batch_ref[...] == gids).astype(jnp.float32)
    sums = lax.dot_general(onehot, h, (((0,), (0,)), ((), ())),
                           preferred_element_type=jnp.float32)
    counts = lax.dot_general(onehot, jnp.ones((N, 1), jnp.float32),
                             (((0,), (0,)), ((), ())),
                             preferred_element_type=jnp.float32)
    pooled = sums / jnp.maximum(counts, 1.0)
    logits = jnp.dot(pooled, wc_ref[...],
                     preferred_element_type=jnp.float32) + bc_ref[...]
    m = jnp.max(logits, axis=1, keepdims=True)
    ex = jnp.exp(logits - m)
    out_ref[...] = ex / jnp.sum(ex, axis=1, keepdims=True)


_tc1 = pl.pallas_call(
    _tc1_body, out_shape=jax.ShapeDtypeStruct((N, D), jnp.float32))
_tc2 = pl.pallas_call(
    _tc2_body, out_shape=jax.ShapeDtypeStruct((N, D), jnp.float32))
_tc3 = pl.pallas_call(
    _tc3_body, out_shape=jax.ShapeDtypeStruct((G, C), jnp.float32))


def kernel(x, edge_index, batch, W1, b1, W2, b2, Wc, bc):
    row = edge_index[0]
    col = edge_index[1]
    col_flat = col
    zeros128 = jnp.zeros((NP, D), jnp.float32)
    ones128 = jnp.ones((K, D), jnp.float32)

    degp = _get_sc_degree()(col_flat, ones128, zeros128)
    y1 = _tc1(x, W1, degp)
    accp1 = _get_sc_gather_scatter()(y1, row, col, zeros128)
    y2 = _tc2(accp1, y1, degp, b1.reshape(1, D), W2)
    accp2 = _get_sc_gather_scatter()(y2, row, col, zeros128)
    return _tc3(accp2, y2, degp, b2.reshape(1, D), batch.reshape(N, 1),
                Wc, bc.reshape(1, C))


# speedup vs baseline: 1.5373x; 1.3002x over previous
import functools

import jax
import jax.numpy as jnp
from jax import lax
from jax.experimental import pallas as pl
from jax.experimental.pallas import tpu as pltpu
from jax.experimental.pallas import tpu_sc as plsc

N = 10000
E = 320000
D = 128
G = 64
C = 16

NC = 2
NS = 16
NW = NC * NS
EP = E // NW
K = 40
NCHUNK = EP // K
NP = 10240
RP = NP // NS

def _make_mesh():
    return plsc.VectorSubcoreMesh(core_axis_name="c", subcore_axis_name="s",
                                  num_cores=NC, num_subcores=NS)


KD = 80
NBD = 5
NGD = EP // KD // NBD


@functools.cache
def _get_sc_degree():
  return pl.kernel(
    _sc_degree_body,
    out_type=jax.ShapeDtypeStruct((NC * NP, D), jnp.float32),
    mesh=_make_mesh(),
    scratch_types=[
        pltpu.VMEM((EP,), jnp.int32),
        pltpu.VMEM((KD, D), jnp.float32),
        [pltpu.SemaphoreType.DMA] * NBD,
        pltpu.VMEM_SHARED((NP, D), jnp.float32),
    ],
  )


def _sc_degree_body(col_hbm, ones_hbm, zeros_hbm, degp_hbm, colv, onesv,
                    sems, dega):
    c = lax.axis_index("c")
    s = lax.axis_index("s")
    base = (c * NS + s) * EP
    pltpu.sync_copy(col_hbm.at[pl.ds(base, EP)], colv)
    pltpu.sync_copy(zeros_hbm.at[pl.ds(s * RP, RP)], dega.at[pl.ds(s * RP, RP)])
    pltpu.sync_copy(ones_hbm, onesv)
    plsc.subcore_barrier()

    def rnd(j, carry):
        for b in range(NBD):
            @pl.when(j > 0)
            def _():
                pltpu.make_async_copy(
                    onesv,
                    dega.at[colv.at[pl.ds((j * NBD + b - NBD) * KD, KD)]],
                    sems[b]).wait()
            pltpu.async_copy(
                onesv, dega.at[colv.at[pl.ds((j * NBD + b) * KD, KD)]],
                sems[b], add=True)
        return carry

    lax.fori_loop(0, NGD, rnd, 0)
    for b in range(NBD):
        pltpu.make_async_copy(
            onesv, dega.at[colv.at[pl.ds(((NGD - 1) * NBD + b) * KD, KD)]],
            sems[b]).wait()
    plsc.subcore_barrier()
    pltpu.sync_copy(dega.at[pl.ds(s * RP, RP)],
                    degp_hbm.at[pl.ds(c * NP + s * RP, RP)])


NB = 5
NG = NCHUNK // NB


@functools.cache
def _get_sc_gather_scatter():
  return pl.kernel(
    _sc_gather_scatter_body,
    out_type=jax.ShapeDtypeStruct((NC * NP, D), jnp.float32),
    mesh=_make_mesh(),
    scratch_types=[
        pltpu.VMEM((EP,), jnp.int32),
        pltpu.VMEM((EP,), jnp.int32),
        [pltpu.VMEM((K, D), jnp.float32)] * NB,
        [pltpu.SemaphoreType.DMA] * NB,
        [pltpu.SemaphoreType.DMA] * NB,
        pltpu.VMEM_SHARED((NP, D), jnp.float32),
    ],
  )


def _sc_gather_scatter_body(y_hbm, row_hbm, col_hbm, zeros_hbm, accp_hbm,
                            idxr, idxc, bufs, sem_g, sem_s, acc):
    c = lax.axis_index("c")
    s = lax.axis_index("s")
    base = (c * NS + s) * EP
    pltpu.sync_copy(row_hbm.at[pl.ds(base, EP)], idxr)
    pltpu.sync_copy(col_hbm.at[pl.ds(base, EP)], idxc)
    pltpu.sync_copy(zeros_hbm.at[pl.ds(s * RP, RP)], acc.at[pl.ds(s * RP, RP)])
    plsc.subcore_barrier()

    def rnd(j, carry):
        gds = []
        for b in range(NB):
            @pl.when(j > 0)
            def _():
                pltpu.make_async_copy(
                    bufs[b],
                    acc.at[idxc.at[pl.ds((j * NB + b - NB) * K, K)]],
                    sem_s[b]).wait()
            gds.append(pltpu.async_copy(
                y_hbm.at[idxr.at[pl.ds((j * NB + b) * K, K)]],
                bufs[b], sem_g[b]))
        for b in range(NB):
            gds[b].wait()
            pltpu.async_copy(bufs[b],
                             acc.at[idxc.at[pl.ds((j * NB + b) * K, K)]],
                             sem_s[b], add=True)
        return carry

    lax.fori_loop(0, NG, rnd, 0)
    for b in range(NB):
        pltpu.make_async_copy(
            bufs[b], acc.at[idxc.at[pl.ds(((NG - 1) * NB + b) * K, K)]],
            sem_s[b]).wait()
    plsc.subcore_barrier()
    pltpu.sync_copy(acc.at[pl.ds(s * RP, RP)],
                    accp_hbm.at[pl.ds(c * NP + s * RP, RP)])


def _dinv_from(degp_ref):
    deg = degp_ref[0:N, 0:1] + degp_ref[NP:NP + N, 0:1] + 1.0
    return lax.rsqrt(deg)


def _tc1_body(x_ref, w1_ref, degp_ref, y_ref):
    dinv = _dinv_from(degp_ref)
    xw = jnp.dot(x_ref[...], w1_ref[...], preferred_element_type=jnp.float32)
    y_ref[...] = xw * dinv


def _tc2_body(accp_ref, y1_ref, degp_ref, b1_ref, w2_ref, y2_ref):
    dinv = _dinv_from(degp_ref)
    agg = accp_ref[0:N, :] + accp_ref[NP:NP + N, :] + y1_ref[...]
    h = jnp.maximum(agg * dinv + b1_ref[...], 0.0)
    y2_ref[...] = jnp.dot(h, w2_ref[...], preferred_element_type=jnp.float32) * dinv


def _tc3_body(accp_ref, y2_ref, degp_ref, b2_ref, batch_ref, wc_ref, bc_ref,
              out_ref):
    dinv = _dinv_from(degp_ref)
    agg = accp_ref[0:N, :] + accp_ref[NP:NP + N, :] + y2_ref[...]
    h = jnp.maximum(agg * dinv + b2_ref[...], 0.0)
    gids = lax.broadcasted_iota(jnp.int32, (N, G), 1)
    onehot = (batch_ref[...] == gids).astype(jnp.float32)
    sums = lax.dot_general(onehot, h, (((0,), (0,)), ((), ())),
                           preferred_element_type=jnp.float32)
    counts = lax.dot_general(onehot, jnp.ones((N, 1), jnp.float32),
                             (((0,), (0,)), ((), ())),
                             preferred_element_type=jnp.float32)
    pooled = sums / jnp.maximum(counts, 1.0)
    logits = jnp.dot(pooled, wc_ref[...],
                     preferred_element_type=jnp.float32) + bc_ref[...]
    m = jnp.max(logits, axis=1, keepdims=True)
    ex = jnp.exp(logits - m)
    out_ref[...] = ex / jnp.sum(ex, axis=1, keepdims=True)


_tc1 = pl.pallas_call(
    _tc1_body, out_shape=jax.ShapeDtypeStruct((N, D), jnp.float32))
_tc2 = pl.pallas_call(
    _tc2_body, out_shape=jax.ShapeDtypeStruct((N, D), jnp.float32))
_tc3 = pl.pallas_call(
    _tc3_body, out_shape=jax.ShapeDtypeStruct((G, C), jnp.float32))


def kernel(x, edge_index, batch, W1, b1, W2, b2, Wc, bc):
    row = edge_index[0]
    col = edge_index[1]
    col_flat = col
    zeros128 = jnp.zeros((NP, D), jnp.float32)
    ones128 = jnp.ones((KD, D), jnp.float32)

    degp = _get_sc_degree()(col_flat, ones128, zeros128)
    y1 = _tc1(x, W1, degp)
    accp1 = _get_sc_gather_scatter()(y1, row, col, zeros128)
    y2 = _tc2(accp1, y1, degp, b1.reshape(1, D), W2)
    accp2 = _get_sc_gather_scatter()(y2, row, col, zeros128)
    return _tc3(accp2, y2, degp, b2.reshape(1, D), batch.reshape(N, 1),
                Wc, bc.reshape(1, C))
